# baseline (speedup 1.0000x reference)
import functools

import jax
import jax.numpy as jnp
from jax import lax
from jax.experimental import pallas as pl
from jax.experimental.pallas import tpu as pltpu
from jax.experimental.pallas import tpu_sc as plsc

_V = 1000000
_D = 64
_NI = 16384
_NJ = 50
_NC, _NS = 2, 16
_NW = _NC * _NS
_IBLK = _NI // _NW
_Q = 128
_QPJ = _IBLK // _Q
_NT = _NJ * _QPJ


@functools.partial(
    pl.kernel,
    out_type=jax.ShapeDtypeStruct((_NJ, _D, _NI), jnp.float32),
    mesh=plsc.VectorSubcoreMesh(core_axis_name="c", subcore_axis_name="s"),
    scratch_types=[
        pltpu.VMEM((_NJ, _IBLK), jnp.int32),
        pltpu.VMEM((_Q,), jnp.int32),
        pltpu.VMEM((_Q,), jnp.int32),
        pltpu.VMEM((_Q,), jnp.int32),
        pltpu.VMEM((_Q,), jnp.int32),
        pltpu.VMEM((_Q,), jnp.int32),
        pltpu.VMEM((_Q,), jnp.int32),
        pltpu.VMEM((_Q,), jnp.int32),
        pltpu.VMEM((_Q,), jnp.int32),
        pltpu.VMEM((_Q, 2 * _D), jnp.float32),
        pltpu.VMEM((_Q, 2 * _D), jnp.float32),
        pltpu.VMEM((_Q, 2 * _D), jnp.float32),
        pltpu.VMEM((_Q, 2 * _D), jnp.float32),
        pltpu.VMEM((_D, _Q), jnp.float32),
        pltpu.VMEM((_D, _Q), jnp.float32),
        pltpu.SemaphoreType.DMA,
        pltpu.SemaphoreType.DMA,
        pltpu.SemaphoreType.DMA,
        pltpu.SemaphoreType.DMA,
        pltpu.SemaphoreType.DMA,
        pltpu.SemaphoreType.DMA,
    ],
    compiler_params=pltpu.CompilerParams(
        use_tc_tiling_on_sc=True,
        needs_layout_passes=False,
        disable_bounds_checks=True,
    ),
)
def _emb_gather(idxT_hbm, pairs_hbm, outq_hbm, idxblk,
                sbuf0, sbuf1, sbuf2, sbuf3, hbuf0, hbuf1, hbuf2, hbuf3,
                gbuf0, gbuf1, gbuf2, gbuf3, qbuf0, qbuf1,
                gs0, gs1, gs2, gs3, ss0, ss1):
    wid = lax.axis_index("s") * _NC + lax.axis_index("c")
    i0 = wid * _IBLK
    sbuf = (sbuf0, sbuf1, sbuf2, sbuf3)
    hbuf = (hbuf0, hbuf1, hbuf2, hbuf3)
    gbuf = (gbuf0, gbuf1, gbuf2, gbuf3)
    qbuf = (qbuf0, qbuf1)
    gs = (gs0, gs1, gs2, gs3)
    ss = (ss0, ss1)
    lane = lax.iota(jnp.int32, 16)

    pltpu.sync_copy(idxT_hbm.at[:, pl.ds(i0, _IBLK)], idxblk)

    def prep_and_fire(t, b):
        j = t // _QPJ
        q = t % _QPJ
        base = q * _Q
        for c in range(_Q // 16):
            v = idxblk[j, pl.ds(base + c * 16, 16)]
            sbuf[b][pl.ds(c * 16, 16)] = lax.shift_right_logical(v, 1)
            hbuf[b][pl.ds(c * 16, 16)] = lax.shift_left(
                lax.bitwise_and(v, 1), 6
            )
        pltpu.async_copy(pairs_hbm.at[sbuf[b]], gbuf[b], gs[b])

    def wait_gather(b):
        pltpu.make_async_copy(pairs_hbm.at[sbuf[b]], gbuf[b], gs[b]).wait()

    def extract_and_store(t, b, sb):
        j = t // _QPJ
        q = t % _QPJ

        @plsc.parallel_loop(0, _Q // 16, unroll=2)
        def chunk_body(c):
            rows = lane + c * 16
            hc = hbuf[b][pl.ds(c * 16, 16)]
            for d in range(_D):
                qbuf[sb][d, pl.ds(c * 16, 16)] = plsc.load_gather(
                    gbuf[b], [rows, hc + d]
                )
        pltpu.async_copy(
            qbuf[sb], outq_hbm.at[j, :, pl.ds(i0 + q * _Q, _Q)], ss[sb]
        )

    def drain_store(sb):
        pltpu.make_async_copy(
            qbuf[sb], outq_hbm.at[0, :, pl.ds(i0, _Q)], ss[sb]
        ).wait()

    prep_and_fire(0, 0)
    prep_and_fire(1, 1)
    prep_and_fire(2, 2)

    def body(p, carry):
        for b in range(4):
            t = 4 * p + b
            pl.when(t + 3 < _NT)(
                lambda b=b, t=t: prep_and_fire(t + 3, (b + 3) % 4)
            )
            wait_gather(b)
            pl.when(t >= 2)(lambda b=b: drain_store(b % 2))
            extract_and_store(t, b, b % 2)
        return carry

    lax.fori_loop(0, _NT // 4, body, 0)
    drain_store(0)
    drain_store(1)


def kernel(input_, weight):
    idxT = input_.T.astype(jnp.int32)
    pairs = weight.reshape(_V // 2, 2 * _D)
    outq = _emb_gather(idxT, pairs)
    return outq.transpose(2, 0, 1)


# speedup vs baseline: 1.4767x; 1.4767x over previous
import functools

import jax
import jax.numpy as jnp
from jax import lax
from jax.experimental import pallas as pl
from jax.experimental.pallas import tpu as pltpu
from jax.experimental.pallas import tpu_sc as plsc

_V = 1000000
_D = 64
_NI = 16384
_NJ = 50
_NC, _NS = 2, 16
_NW = _NC * _NS
_IBLK = _NI // _NW
_Q = 128
_QPJ = _IBLK // _Q
_NT = _NJ * _QPJ


@functools.partial(
    pl.kernel,
    out_type=jax.ShapeDtypeStruct((_NJ, _D, _NI), jnp.float32),
    mesh=plsc.VectorSubcoreMesh(core_axis_name="c", subcore_axis_name="s"),
    scratch_types=[
        pltpu.VMEM((_NJ, _IBLK), jnp.int32),
        pltpu.VMEM((_Q,), jnp.int32),
        pltpu.VMEM((_Q,), jnp.int32),
        pltpu.VMEM((_Q,), jnp.int32),
        pltpu.VMEM((_Q,), jnp.int32),
        pltpu.VMEM((_Q,), jnp.int32),
        pltpu.VMEM((_Q,), jnp.int32),
        pltpu.VMEM((_Q,), jnp.int32),
        pltpu.VMEM((_Q,), jnp.int32),
        pltpu.VMEM((_Q, 2 * _D), jnp.float32),
        pltpu.VMEM((_Q, 2 * _D), jnp.float32),
        pltpu.VMEM((_Q, 2 * _D), jnp.float32),
        pltpu.VMEM((_Q, 2 * _D), jnp.float32),
        pltpu.VMEM((_D, _Q), jnp.float32),
        pltpu.VMEM((_D, _Q), jnp.float32),
        pltpu.SemaphoreType.DMA,
        pltpu.SemaphoreType.DMA,
        pltpu.SemaphoreType.DMA,
        pltpu.SemaphoreType.DMA,
        pltpu.SemaphoreType.DMA,
        pltpu.SemaphoreType.DMA,
    ],
    compiler_params=pltpu.CompilerParams(
        use_tc_tiling_on_sc=True,
        needs_layout_passes=False,
        disable_bounds_checks=True,
    ),
)
def _emb_gather(idxT_hbm, pairs_hbm, outq_hbm, idxblk,
                sbuf0, sbuf1, sbuf2, sbuf3, hbuf0, hbuf1, hbuf2, hbuf3,
                gbuf0, gbuf1, gbuf2, gbuf3, qbuf0, qbuf1,
                gs0, gs1, gs2, gs3, ss0, ss1):
    wid = lax.axis_index("s") * _NC + lax.axis_index("c")
    i0 = wid * _IBLK
    sbuf = (sbuf0, sbuf1, sbuf2, sbuf3)
    hbuf = (hbuf0, hbuf1, hbuf2, hbuf3)
    gbuf = (gbuf0, gbuf1, gbuf2, gbuf3)
    qbuf = (qbuf0, qbuf1)
    gs = (gs0, gs1, gs2, gs3)
    ss = (ss0, ss1)
    lane = lax.iota(jnp.int32, 16)

    pltpu.sync_copy(idxT_hbm.at[:, pl.ds(i0, _IBLK)], idxblk)

    def prep_and_fire(t, b):
        j = t // _QPJ
        q = t % _QPJ
        base = q * _Q
        for c in range(_Q // 16):
            v = idxblk[j, pl.ds(base + c * 16, 16)]
            sbuf[b][pl.ds(c * 16, 16)] = lax.shift_right_logical(v, 1)
            hbuf[b][pl.ds(c * 16, 16)] = lax.shift_left(
                lax.bitwise_and(v, 1), 6
            )
        pltpu.async_copy(pairs_hbm.at[sbuf[b]], gbuf[b], gs[b])

    def wait_gather(b):
        pltpu.make_async_copy(pairs_hbm.at[sbuf[b]], gbuf[b], gs[b]).wait()

    def extract_and_store(t, b, sb):
        j = t // _QPJ
        q = t % _QPJ

        @plsc.parallel_loop(0, _Q // 16, unroll=2)
        def chunk_body(c):
            rows = lane + c * 16
            hc = hbuf[b][pl.ds(c * 16, 16)]
            for d in range(_D):
                dvec = lax.bitwise_and(lane + d, _D - 1)
                val = plsc.load_gather(gbuf[b], [rows, hc + dvec])
                plsc.store_scatter(qbuf[sb], [dvec, rows], val)
        pltpu.async_copy(
            qbuf[sb], outq_hbm.at[j, :, pl.ds(i0 + q * _Q, _Q)], ss[sb]
        )

    def drain_store(sb):
        pltpu.make_async_copy(
            qbuf[sb], outq_hbm.at[0, :, pl.ds(i0, _Q)], ss[sb]
        ).wait()

    prep_and_fire(0, 0)
    prep_and_fire(1, 1)
    prep_and_fire(2, 2)

    def body(p, carry):
        for b in range(4):
            t = 4 * p + b
            pl.when(t + 3 < _NT)(
                lambda b=b, t=t: prep_and_fire(t + 3, (b + 3) % 4)
            )
            wait_gather(b)
            pl.when(t >= 2)(lambda b=b: drain_store(b % 2))
            extract_and_store(t, b, b % 2)
        return carry

    lax.fori_loop(0, _NT // 4, body, 0)
    drain_store(0)
    drain_store(1)


def kernel(input_, weight):
    idxT = input_.T.astype(jnp.int32)
    pairs = weight.reshape(_V // 2, 2 * _D)
    outq = _emb_gather(idxT, pairs)
    return outq.transpose(2, 0, 1)


# speedup vs baseline: 1.5577x; 1.0549x over previous
import functools

import jax
import jax.numpy as jnp
from jax import lax
from jax.experimental import pallas as pl
from jax.experimental.pallas import tpu as pltpu
from jax.experimental.pallas import tpu_sc as plsc

_V = 1000000
_D = 64
_NI = 16384
_NJ = 50
_NC, _NS = 2, 16
_NW = _NC * _NS
_IBLK = _NI // _NW
_Q = 128
_QPJ = _IBLK // _Q
_NT = _NJ * _QPJ


@functools.partial(
    pl.kernel,
    out_type=jax.ShapeDtypeStruct((_NJ, _D, _NI), jnp.float32),
    mesh=plsc.VectorSubcoreMesh(core_axis_name="c", subcore_axis_name="s"),
    scratch_types=[
        pltpu.VMEM((_NJ, _IBLK), jnp.int32),
        pltpu.VMEM((_Q,), jnp.int32),
        pltpu.VMEM((_Q,), jnp.int32),
        pltpu.VMEM((_Q,), jnp.int32),
        pltpu.VMEM((_Q,), jnp.int32),
        pltpu.VMEM((_Q,), jnp.int32),
        pltpu.VMEM((_Q,), jnp.int32),
        pltpu.VMEM((_Q,), jnp.int32),
        pltpu.VMEM((_Q,), jnp.int32),
        pltpu.VMEM((_Q, 2 * _D), jnp.float32),
        pltpu.VMEM((_Q, 2 * _D), jnp.float32),
        pltpu.VMEM((_Q, 2 * _D), jnp.float32),
        pltpu.VMEM((_Q, 2 * _D), jnp.float32),
        pltpu.VMEM((_D, _Q), jnp.float32),
        pltpu.VMEM((_D, _Q), jnp.float32),
        pltpu.SemaphoreType.DMA,
        pltpu.SemaphoreType.DMA,
        pltpu.SemaphoreType.DMA,
        pltpu.SemaphoreType.DMA,
        pltpu.SemaphoreType.DMA,
        pltpu.SemaphoreType.DMA,
    ],
    compiler_params=pltpu.CompilerParams(
        use_tc_tiling_on_sc=True,
        needs_layout_passes=False,
        disable_bounds_checks=True,
    ),
)
def _emb_gather(idxT_hbm, pairs_hbm, outq_hbm, idxblk,
                sbuf0, sbuf1, sbuf2, sbuf3, hbuf0, hbuf1, hbuf2, hbuf3,
                gbuf0, gbuf1, gbuf2, gbuf3, qbuf0, qbuf1,
                gs0, gs1, gs2, gs3, ss0, ss1):
    wid = lax.axis_index("s") * _NC + lax.axis_index("c")
    i0 = wid * _IBLK
    sbuf = (sbuf0, sbuf1, sbuf2, sbuf3)
    hbuf = (hbuf0, hbuf1, hbuf2, hbuf3)
    gbuf = (gbuf0, gbuf1, gbuf2, gbuf3)
    qbuf = (qbuf0, qbuf1)
    gs = (gs0, gs1, gs2, gs3)
    ss = (ss0, ss1)
    lane = lax.iota(jnp.int32, 16)

    pltpu.sync_copy(idxT_hbm.at[:, pl.ds(i0, _IBLK)], idxblk)

    def prep_and_fire(t, b):
        j = t // _QPJ
        q = t % _QPJ
        base = q * _Q
        for c in range(_Q // 16):
            v = idxblk[j, pl.ds(base + c * 16, 16)]
            sbuf[b][pl.ds(c * 16, 16)] = lax.shift_right_logical(v, 1)
            hbuf[b][pl.ds(c * 16, 16)] = lax.shift_left(
                lax.bitwise_and(v, 1), 6
            )
        pltpu.async_copy(pairs_hbm.at[sbuf[b]], gbuf[b], gs[b])

    def wait_gather(b):
        pltpu.make_async_copy(pairs_hbm.at[sbuf[b]], gbuf[b], gs[b]).wait()

    def extract_and_store(t, b, sb):
        j = t // _QPJ
        q = t % _QPJ

        @plsc.parallel_loop(0, _Q // 16, unroll=2)
        def chunk_body(c):
            rows = lane + c * 16
            hc = hbuf[b][pl.ds(c * 16, 16)]
            for d in range(_D):
                dvec = lax.bitwise_and(lane + d, _D - 1)
                val = plsc.load_gather(gbuf[b], [rows, hc + dvec])
                plsc.store_scatter(qbuf[sb], [dvec, rows], val)
        pltpu.async_copy(
            qbuf[sb], outq_hbm.at[j, :, pl.ds(i0 + q * _Q, _Q)], ss[sb]
        )

    def drain_store(sb):
        pltpu.make_async_copy(
            qbuf[sb], outq_hbm.at[0, :, pl.ds(i0, _Q)], ss[sb]
        ).wait()

    prep_and_fire(0, 0)
    prep_and_fire(1, 1)
    prep_and_fire(2, 2)

    def body(p, carry):
        for b in range(4):
            t = 4 * p + b
            pl.when(t + 3 < _NT)(
                lambda b=b, t=t: prep_and_fire(t + 3, (b + 3) % 4)
            )
            wait_gather(b)
            pl.when(t >= 2)(lambda b=b: drain_store(b % 2))
            extract_and_store(t, b, b % 2)
        return carry

    lax.fori_loop(0, _NT // 4, body, 0)
    drain_store(0)
    drain_store(1)


_NB = _V // 128
_NBW = 246


@functools.partial(
    pl.kernel,
    out_type=jax.ShapeDtypeStruct((_V // 2, 2 * _D), jnp.float32),
    mesh=plsc.VectorSubcoreMesh(core_axis_name="c", subcore_axis_name="s"),
    scratch_types=[
        pltpu.VMEM((_D, 128), jnp.float32),
        pltpu.VMEM((_D, 128), jnp.float32),
        pltpu.VMEM((_D, 128), jnp.float32),
        pltpu.VMEM((_D, 128), jnp.float32),
        pltpu.SemaphoreType.DMA,
        pltpu.SemaphoreType.DMA,
        pltpu.SemaphoreType.DMA,
        pltpu.SemaphoreType.DMA,
    ],
    compiler_params=pltpu.CompilerParams(
        use_tc_tiling_on_sc=True,
        needs_layout_passes=False,
        disable_bounds_checks=True,
    ),
)
def _pair_pack(wT_hbm, wtail_hbm, pairs_hbm, tb0, tb1, pb0, pb1,
               ls0, ls1, ps0, ps1):
    wid = lax.axis_index("s") * _NC + lax.axis_index("c")
    tb = (tb0, tb1)
    pb = (pb0, pb1)
    ls = (ls0, ls1)
    ps = (ps0, ps1)
    lane = lax.iota(jnp.int32, 16)

    def load_block(t, u):
        b = wid + _NW * t

        def fire():
            pltpu.async_copy(wT_hbm.at[:, pl.ds(b * 128, 128)], tb[u], ls[u])

        pl.when(b < _NB)(fire)

    def transpose_block(u, ncc):
        @plsc.parallel_loop(0, ncc, unroll=2)
        def cc_body(cc):
            ccvec = lane + cc * 16
            prow = lax.shift_right_logical(lane, 1) + cc * 8
            for d in range(_D):
                dvec = lax.bitwise_and(lane + d, _D - 1)
                pcol = lax.bitwise_and(lane, 1) * _D + dvec
                val = plsc.load_gather(tb[u], [dvec, ccvec])
                plsc.store_scatter(pb[u], [prow, pcol], val)

    def process_block(t, u):
        b = wid + _NW * t

        def do():
            pltpu.make_async_copy(
                wT_hbm.at[:, pl.ds(0, 128)], tb[u], ls[u]
            ).wait()
            transpose_block(u, 8)
            pltpu.async_copy(
                pb[u], pairs_hbm.at[pl.ds(b * _D, _D)], ps[u]
            )

        pl.when(b < _NB)(do)

    def drain_store(u):
        pltpu.make_async_copy(
            pb[u], pairs_hbm.at[pl.ds(0, _D)], ps[u]
        ).wait()

    load_block(0, 0)

    def body(t2, carry):
        for uu in range(2):
            t = 2 * t2 + uu
            load_block(t + 1, 1 - uu)
            pl.when(t >= 2)(
                lambda uu=uu, t=t: pl.when(wid + _NW * (t - 2) < _NB)(
                    lambda: drain_store(uu)
                )
            )
            process_block(t, uu)
        return carry

    lax.fori_loop(0, _NBW // 2, body, 0)
    pl.when(wid + _NW * (_NBW - 2) < _NB)(lambda: drain_store(0))
    pl.when(wid + _NW * (_NBW - 1) < _NB)(lambda: drain_store(1))

    def tail():
        pltpu.sync_copy(wtail_hbm, tb0.at[pl.ds(0, _D // 2)])
        pltpu.sync_copy(
            tb0.at[pl.ds(0, _D // 2)], pairs_hbm.at[pl.ds(_NB * _D, _D // 2)]
        )

    pl.when(wid == 0)(tail)


def kernel(input_, weight):
    idxT = input_.T.astype(jnp.int32)
    wtail = weight[_NB * 128:].reshape(_D // 2, 2 * _D)
    pairs = _pair_pack(weight.T, wtail)
    outq = _emb_gather(idxT, pairs)
    return outq.transpose(2, 0, 1)


# speedup vs baseline: 1.9018x; 1.2209x over previous
import functools

import jax
import jax.numpy as jnp
from jax import lax
from jax.experimental import pallas as pl
from jax.experimental.pallas import tpu as pltpu
from jax.experimental.pallas import tpu_sc as plsc

_V = 1000000
_D = 64
_NI = 16384
_NJ = 50
_NC, _NS = 2, 16
_NW = _NC * _NS
_IBLK = _NI // _NW
_Q = 128
_QPJ = _IBLK // _Q
_NT = _NJ * _QPJ


@functools.partial(
    pl.kernel,
    out_type=jax.ShapeDtypeStruct((_NJ, _D, _NI), jnp.float32),
    mesh=plsc.VectorSubcoreMesh(core_axis_name="c", subcore_axis_name="s"),
    scratch_types=[
        pltpu.VMEM((_NJ, _IBLK), jnp.int32),
        pltpu.VMEM((_Q,), jnp.int32),
        pltpu.VMEM((_Q,), jnp.int32),
        pltpu.VMEM((_Q,), jnp.int32),
        pltpu.VMEM((_Q,), jnp.int32),
        pltpu.VMEM((_Q,), jnp.int32),
        pltpu.VMEM((_Q,), jnp.int32),
        pltpu.VMEM((_Q,), jnp.int32),
        pltpu.VMEM((_Q,), jnp.int32),
        pltpu.VMEM((_Q, 2 * _D), jnp.float32),
        pltpu.VMEM((_Q, 2 * _D), jnp.float32),
        pltpu.VMEM((_Q, 2 * _D), jnp.float32),
        pltpu.VMEM((_Q, 2 * _D), jnp.float32),
        pltpu.VMEM((_D, _Q), jnp.float32),
        pltpu.VMEM((_D, _Q), jnp.float32),
        pltpu.SemaphoreType.DMA,
        pltpu.SemaphoreType.DMA,
        pltpu.SemaphoreType.DMA,
        pltpu.SemaphoreType.DMA,
        pltpu.SemaphoreType.DMA,
        pltpu.SemaphoreType.DMA,
    ],
    compiler_params=pltpu.CompilerParams(
        use_tc_tiling_on_sc=True,
        needs_layout_passes=False,
        disable_bounds_checks=True,
    ),
)
def _emb_gather(idxT_hbm, pairs_hbm, outq_hbm, idxblk,
                sbuf0, sbuf1, sbuf2, sbuf3, hbuf0, hbuf1, hbuf2, hbuf3,
                gbuf0, gbuf1, gbuf2, gbuf3, qbuf0, qbuf1,
                gs0, gs1, gs2, gs3, ss0, ss1):
    wid = lax.axis_index("s") * _NC + lax.axis_index("c")
    i0 = wid * _IBLK
    sbuf = (sbuf0, sbuf1, sbuf2, sbuf3)
    hbuf = (hbuf0, hbuf1, hbuf2, hbuf3)
    gbuf = (gbuf0, gbuf1, gbuf2, gbuf3)
    qbuf = (qbuf0, qbuf1)
    gs = (gs0, gs1, gs2, gs3)
    ss = (ss0, ss1)
    lane = lax.iota(jnp.int32, 16)

    pltpu.sync_copy(idxT_hbm.at[:, pl.ds(i0, _IBLK)], idxblk)

    def prep_and_fire(t, b):
        j = t // _QPJ
        q = t % _QPJ
        base = q * _Q
        for c in range(_Q // 16):
            v = idxblk[j, pl.ds(base + c * 16, 16)]
            sbuf[b][pl.ds(c * 16, 16)] = lax.shift_right_logical(v, 1)
            hbuf[b][pl.ds(c * 16, 16)] = lax.shift_left(
                lax.bitwise_and(v, 1), 6
            )
        pltpu.async_copy(pairs_hbm.at[sbuf[b]], gbuf[b], gs[b])

    def wait_gather(b):
        pltpu.make_async_copy(pairs_hbm.at[sbuf[b]], gbuf[b], gs[b]).wait()

    def extract_and_store(t, b, sb):
        j = t // _QPJ
        q = t % _QPJ

        @plsc.parallel_loop(0, _Q // 16, unroll=2)
        def chunk_body(c):
            rows = lane + c * 16
            hc = hbuf[b][pl.ds(c * 16, 16)]
            for d in range(_D):
                dvec = lax.bitwise_and(lane + d, _D - 1)
                val = plsc.load_gather(gbuf[b], [rows, hc + dvec])
                plsc.store_scatter(qbuf[sb], [dvec, rows], val)
        pltpu.async_copy(
            qbuf[sb], outq_hbm.at[j, :, pl.ds(i0 + q * _Q, _Q)], ss[sb]
        )

    def drain_store(sb):
        pltpu.make_async_copy(
            qbuf[sb], outq_hbm.at[0, :, pl.ds(i0, _Q)], ss[sb]
        ).wait()

    prep_and_fire(0, 0)
    prep_and_fire(1, 1)
    prep_and_fire(2, 2)

    def body(p, carry):
        for b in range(4):
            t = 4 * p + b
            pl.when(t + 3 < _NT)(
                lambda b=b, t=t: prep_and_fire(t + 3, (b + 3) % 4)
            )
            wait_gather(b)
            pl.when(t >= 2)(lambda b=b: drain_store(b % 2))
            extract_and_store(t, b, b % 2)
        return carry

    lax.fori_loop(0, _NT // 4, body, 0)
    drain_store(0)
    drain_store(1)


_BW = 256
_NB = _V // _BW
_NBW = 124


@functools.partial(
    pl.kernel,
    out_type=jax.ShapeDtypeStruct((_V // 2, 2 * _D), jnp.float32),
    mesh=plsc.VectorSubcoreMesh(core_axis_name="c", subcore_axis_name="s"),
    scratch_types=[
        pltpu.VMEM((_D, _BW), jnp.float32),
        pltpu.VMEM((_D, _BW), jnp.float32),
        pltpu.VMEM((_BW // 2, 128), jnp.float32),
        pltpu.VMEM((_BW // 2, 128), jnp.float32),
        pltpu.SemaphoreType.DMA,
        pltpu.SemaphoreType.DMA,
        pltpu.SemaphoreType.DMA,
        pltpu.SemaphoreType.DMA,
    ],
    compiler_params=pltpu.CompilerParams(
        use_tc_tiling_on_sc=True,
        needs_layout_passes=False,
        disable_bounds_checks=True,
    ),
)
def _pair_pack(wT_hbm, wtail_hbm, pairs_hbm, tb0, tb1, pb0, pb1,
               ls0, ls1, ps0, ps1):
    wid = lax.axis_index("s") * _NC + lax.axis_index("c")
    tb = (tb0, tb1)
    pb = (pb0, pb1)
    ls = (ls0, ls1)
    ps = (ps0, ps1)
    lane = lax.iota(jnp.int32, 16)

    def load_block(t, u):
        b = wid + _NW * t

        def fire():
            pltpu.async_copy(wT_hbm.at[:, pl.ds(b * _BW, _BW)], tb[u], ls[u])

        pl.when(b < _NB)(fire)

    def transpose_block(u, ncc):
        @plsc.parallel_loop(0, ncc, unroll=2)
        def cc_body(cc):
            ccvec = lane + cc * 16
            prow = lax.shift_right_logical(lane, 1) + cc * 8
            for d in range(_D):
                dvec = lax.bitwise_and(lane + d, _D - 1)
                pcol = lax.bitwise_and(lane, 1) * _D + dvec
                val = plsc.load_gather(tb[u], [dvec, ccvec])
                plsc.store_scatter(pb[u], [prow, pcol], val)

    def process_block(t, u):
        b = wid + _NW * t

        def do():
            pltpu.make_async_copy(
                wT_hbm.at[:, pl.ds(0, _BW)], tb[u], ls[u]
            ).wait()
            transpose_block(u, _BW // 16)
            pltpu.async_copy(
                pb[u], pairs_hbm.at[pl.ds(b * (_BW // 2), _BW // 2)], ps[u]
            )

        pl.when(b < _NB)(do)

    def drain_store(u):
        pltpu.make_async_copy(
            pb[u], pairs_hbm.at[pl.ds(0, _BW // 2)], ps[u]
        ).wait()

    load_block(0, 0)

    def body(t2, carry):
        for uu in range(2):
            t = 2 * t2 + uu
            load_block(t + 1, 1 - uu)
            pl.when(t >= 2)(
                lambda uu=uu, t=t: pl.when(wid + _NW * (t - 2) < _NB)(
                    lambda: drain_store(uu)
                )
            )
            process_block(t, uu)
        return carry

    lax.fori_loop(0, _NBW // 2, body, 0)
    pl.when(wid + _NW * (_NBW - 2) < _NB)(lambda: drain_store(0))
    pl.when(wid + _NW * (_NBW - 1) < _NB)(lambda: drain_store(1))

    def tail():
        pltpu.sync_copy(wtail_hbm, pb0.at[pl.ds(0, _D // 2)])
        pltpu.sync_copy(
            pb0.at[pl.ds(0, _D // 2)], pairs_hbm.at[pl.ds(_NB * (_BW // 2), _D // 2)]
        )

    pl.when(wid == 0)(tail)


def kernel(input_, weight):
    idxT = input_.T.astype(jnp.int32)
    wtail = weight[_NB * _BW:].reshape(_D // 2, 2 * _D)
    pairs = _pair_pack(weight.T, wtail)
    outq = _emb_gather(idxT, pairs)
    return outq.transpose(2, 0, 1)
